# Initial kernel scaffold; baseline (speedup 1.0000x reference)
import jax
import jax.numpy as jnp
from jax.experimental import pallas as pl


def kernel(userIdx, itemIdx, rows, cols, L_vals, uEmbd, iEmbd, W1a, b1a, W1b, b1b, W2a, b2a, W2b, b2b, T1w, T1b, T2w, T2b, T3w, T3b):
    raise NotImplementedError("write your pallas kernel here")


# speedup vs baseline: 1.2990x; 1.2990x over previous
import functools

import jax
import jax.numpy as jnp
from jax import lax
from jax.experimental import pallas as pl
from jax.experimental.pallas import tpu as pltpu
from jax.experimental.pallas import tpu_sc as plsc

NC = 2
NS = 16
LN = 16
CH = 128
DG = 128


def _make_gather_scale(n_nodes, w_scale, e_pad):
  e_side = e_pad // NC
  ept = e_side // NS
  nchunk = ept // CH
  assert nchunk * CH == ept
  gs = w_scale // LN
  mesh = plsc.VectorSubcoreMesh(core_axis_name="c", subcore_axis_name="s",
                                num_cores=NC, num_subcores=NS)

  @functools.partial(
      pl.kernel,
      out_type=jax.ShapeDtypeStruct((e_pad, DG), jnp.float32),
      mesh=mesh,
      scratch_types=[
          pltpu.VMEM((CH,), jnp.int32),
          pltpu.VMEM((CH,), jnp.float32),
          pltpu.VMEM((CH, DG), jnp.float32),
          pltpu.SemaphoreType.DMA,
      ],
  )
  def gsc(feat, colsp, valsp, out, colb, valb, gb, sem):
    c = lax.axis_index("c")
    s = lax.axis_index("s")
    ebase = c * e_side + s * ept

    def chunk(i, carry):
      b = ebase + i * CH
      pltpu.sync_copy(colsp.at[pl.ds(b, CH)], colb)
      pltpu.sync_copy(valsp.at[pl.ds(b, CH)], valb)
      pltpu.async_copy(feat.at[colb], gb, sem).wait()

      pltpu.sync_copy(gb, out.at[pl.ds(b, CH)])
      return carry
    lax.fori_loop(0, nchunk, chunk, 0)

  return gsc


def _make_gather3(n_nodes, n_idx):
  per_w = n_idx // (NC * NS)
  nchunk = per_w // CH
  assert nchunk * CH == per_w
  mesh = plsc.VectorSubcoreMesh(core_axis_name="c", subcore_axis_name="s",
                                num_cores=NC, num_subcores=NS)

  @functools.partial(
      pl.kernel,
      out_type=tuple(jax.ShapeDtypeStruct((n_idx, DG), jnp.float32)
                     for _ in range(3)),
      mesh=mesh,
      scratch_types=[
          pltpu.VMEM((CH,), jnp.int32),
          pltpu.VMEM((CH, DG), jnp.float32),
          pltpu.SemaphoreType.DMA,
      ],
  )
  def gat(idx, t0, t1, t2, o0, o1, o2, idxb, buf, sem):
    c = lax.axis_index("c")
    s = lax.axis_index("s")
    wid = s * NC + c

    def chunk(i, carry):
      base = wid * per_w + i * CH
      pltpu.sync_copy(idx.at[pl.ds(base, CH)], idxb)
      for tab, o in ((t0, o0), (t1, o1), (t2, o2)):
        pltpu.async_copy(tab.at[idxb], buf, sem).wait()
        pltpu.sync_copy(buf, o.at[pl.ds(base, CH)])
      return carry
    lax.fori_loop(0, nchunk, chunk, 0)

  return gat


def _tc_scale(upd, vals):
  m = upd.shape[0]
  blk = 8192
  grid = m // blk
  assert grid * blk == m

  def body(u_r, v_r, o_r):
    o_r[...] = u_r[...] * v_r[...]

  return pl.pallas_call(
      body,
      grid=(grid,),
      in_specs=[pl.BlockSpec((blk, DG), lambda i: (i, 0)),
                pl.BlockSpec((blk, 1), lambda i: (i, 0))],
      out_specs=pl.BlockSpec((blk, DG), lambda i: (i, 0)),
      out_shape=jax.ShapeDtypeStruct((m, DG), jnp.float32),
  )(upd, vals.reshape(-1, 1))


def _dg(x, w):
  return lax.dot_general(x, w, (((1,), (1,)), ((), ())),
                         preferred_element_type=jnp.float32)


def _tc_fused(lf, f, wa, wb, ba):
  n = f.shape[0]
  blk = 2000
  grid = n // blk

  def body(lf_r, f_r, wa_r, wb_r, ba_r, p1_r, iw_r):
    l, x = lf_r[...], f_r[...]
    p1_r[...] = _dg(l + x, wa_r[...]) + ba_r[...]
    iw_r[...] = _dg(l * x, wb_r[...])

  full = lambda a: pl.BlockSpec(a.shape, lambda i: (0, 0))
  row = pl.BlockSpec((blk, DG), lambda i: (i, 0))
  return pl.pallas_call(
      body,
      grid=(grid,),
      in_specs=[row, row, full(wa), full(wb), full(ba)],
      out_specs=[row, row],
      out_shape=[jax.ShapeDtypeStruct((n, DG), jnp.float32)] * 2,
  )(lf, f, wa, wb, ba)


def _tc_relu_add(p1, s2, b):
  n = p1.shape[0]
  blk = 2000
  grid = n // blk

  def body(p_r, s_r, b_r, o_r):
    o_r[...] = jnp.maximum(p_r[...] + s_r[...] + b_r[...], 0.0)

  full = lambda a: pl.BlockSpec(a.shape, lambda i: (0, 0))
  row = pl.BlockSpec((blk, DG), lambda i: (i, 0))
  return pl.pallas_call(
      body,
      grid=(grid,),
      in_specs=[row, row, full(b)],
      out_specs=row,
      out_shape=jax.ShapeDtypeStruct((n, DG), jnp.float32),
  )(p1, s2, b)


def _tc_mlp(gs, wu, wi, b1, t2w, b2, t3p):
  batch = gs[0].shape[0] // 2
  blk = 512
  nblk = batch // blk

  def body(g0u, g1u, g2u, g0i, g1i, g2i,
           wu0, wu1, wu2, wi0, wi1, wi2,
           b1_r, t2_r, b2_r, t3_r, out_r):
    h = b1_r[...] + _dg(g0u[...], wu0[...]) + _dg(g1u[...], wu1[...]) \
        + _dg(g2u[...], wu2[...]) \
        + _dg(g0i[...], wi0[...]) + _dg(g1i[...], wi1[...]) \
        + _dg(g2i[...], wi2[...])
    h = jnp.maximum(h, 0.0)
    h2 = _dg(h, t2_r[...]) + b2_r[...]
    out_r[...] = jnp.dot(h2, t3_r[...],
                         preferred_element_type=jnp.float32)

  full = lambda a: pl.BlockSpec(a.shape, lambda i: (0, 0))
  rowu = lambda a: pl.BlockSpec((blk, a.shape[1]), lambda i: (i, 0))
  rowi = lambda a: pl.BlockSpec((blk, a.shape[1]), lambda i: (i + nblk, 0))
  return pl.pallas_call(
      body,
      grid=(nblk,),
      in_specs=[rowu(g) for g in gs] + [rowi(g) for g in gs]
               + [full(w) for w in wu] + [full(w) for w in wi]
               + [full(b1), full(t2w), full(b2), full(t3p)],
      out_specs=pl.BlockSpec((blk, DG), lambda i: (i, 0)),
      out_shape=jax.ShapeDtypeStruct((batch, DG), jnp.float32),
  )(*gs, *gs, *wu, *wi, b1, t2w, b2, t3p)


def _pad2(a, r, c):
  return jnp.pad(a, ((0, r - a.shape[0]), (0, c - a.shape[1])))


def kernel(userIdx, itemIdx, rows, cols, L_vals, uEmbd, iEmbd,
           W1a, b1a, W1b, b1b, W2a, b2a, W2b, b2b,
           T1w, T1b, T2w, T2b, T3w, T3b):
  n_u, d0 = uEmbd.shape
  n_i = iEmbd.shape[0]
  n = n_u + n_i
  assert n_u == n_i
  e = rows.shape[0]
  eh = e // 2

  ept = -(-(eh // NS) // CH) * CH
  e_side = ept * NS
  pad = e_side - eh
  r32 = rows.astype(jnp.int32)
  c32 = cols.astype(jnp.int32)
  v32 = L_vals.astype(jnp.float32)
  zi = jnp.zeros((pad,), jnp.int32)
  zf = jnp.zeros((pad,), jnp.float32)
  colsp = jnp.concatenate([c32[:eh], zi, c32[eh:], zi])
  rowsp = jnp.concatenate([r32[:eh], zi, r32[eh:], zi])
  valsp = jnp.concatenate([v32[:eh], zf, v32[eh:], zf])

  d1 = W1a.shape[0]
  d2 = W2a.shape[0]
  d0p = -(-d0 // LN) * LN
  d1p = -(-d1 // LN) * LN
  dA = 64

  f0p = _pad2(jnp.concatenate([uEmbd, iEmbd], axis=0), n, DG)

  gs112 = _make_gather_scale(n, d0p, 2 * e_side)
  gs80 = _make_gather_scale(n, d1p, 2 * e_side)
  gs64 = _make_gather_scale(n, dA, 2 * e_side)

  def spmm(gsc, feat):
    upd = _tc_scale(gsc(feat, colsp, valsp), valsp)
    return jnp.zeros((n, DG), jnp.float32).at[rowsp].add(upd)

  bpad = lambda b: jnp.pad(b, (0, DG - b.shape[0])).reshape(1, -1)

  lf = spmm(gs112, f0p)
  p1, iw = _tc_fused(lf, f0p, _pad2(W1a, DG, DG), _pad2(W1b, DG, DG),
                     bpad(b1a))
  s2 = spmm(gs80, iw)
  f2 = _tc_relu_add(p1, s2, bpad(b1b))

  lf2 = spmm(gs80, f2)
  p12, iw2 = _tc_fused(lf2, f2, _pad2(W2a, DG, DG), _pad2(W2b, DG, DG),
                       bpad(b2a))
  s22 = spmm(gs64, iw2)
  f3 = _tc_relu_add(p12, s22, bpad(b2b))

  idx = jnp.concatenate([userIdx, itemIdx]).astype(jnp.int32)
  g0, g2, g3 = _make_gather3(n, idx.shape[0])(idx, f0p, f2, f3)

  demb = d0 + d1 + d2

  def t1split(off):
    a = _pad2(T1w[:, off:off + d0], T1b.shape[0], DG)
    b = _pad2(T1w[:, off + d0:off + d0 + d1], T1b.shape[0], DG)
    c2 = _pad2(T1w[:, off + d0 + d1:off + demb], T1b.shape[0], DG)
    return a, b, c2

  t3p = _pad2(T3w.reshape(-1, 1), T2w.shape[0], DG)
  y = _tc_mlp((g0, g2, g3), t1split(0), t1split(demb),
              T1b.reshape(1, -1), T2w, T2b.reshape(1, -1), t3p)
  return y[:, 0] + T3b[0]


# speedup vs baseline: 1.4787x; 1.1384x over previous
import functools

import jax
import jax.numpy as jnp
from jax import lax
from jax.experimental import pallas as pl
from jax.experimental.pallas import tpu as pltpu
from jax.experimental.pallas import tpu_sc as plsc

NC = 2
NS = 16
LN = 16
CH = 128
DG = 128


def _make_gather_scale(n_nodes, w_scale, e_pad):
  e_side = e_pad // NC
  ept = e_side // NS
  nchunk = ept // CH
  assert nchunk * CH == ept
  gs = w_scale // LN
  mesh = plsc.VectorSubcoreMesh(core_axis_name="c", subcore_axis_name="s",
                                num_cores=NC, num_subcores=NS)

  @functools.partial(
      pl.kernel,
      out_type=jax.ShapeDtypeStruct((e_pad, DG), jnp.float32),
      mesh=mesh,
      scratch_types=[
          pltpu.VMEM((CH,), jnp.int32),
          pltpu.VMEM((CH,), jnp.float32),
          pltpu.VMEM((CH, DG), jnp.float32),
          pltpu.SemaphoreType.DMA,
      ],
  )
  def gsc(feat, colsp, valsp, out, colb, valb, gb, sem):
    c = lax.axis_index("c")
    s = lax.axis_index("s")
    ebase = c * e_side + s * ept

    def chunk(i, carry):
      b = ebase + i * CH
      pltpu.sync_copy(colsp.at[pl.ds(b, CH)], colb)
      pltpu.sync_copy(valsp.at[pl.ds(b, CH)], valb)
      pltpu.async_copy(feat.at[colb], gb, sem).wait()

      for g16 in range(CH // LN):
        eb = g16 * LN
        vv = valb[pl.ds(eb, LN)]
        for l in range(LN):
          v = lax.broadcast(vv[l], (LN,))
          for g in range(gs):
            gb[eb + l, pl.ds(g * LN, LN)] = gb[eb + l, pl.ds(g * LN, LN)] * v
      pltpu.sync_copy(gb, out.at[pl.ds(b, CH)])
      return carry
    lax.fori_loop(0, nchunk, chunk, 0)

  return gsc


def _make_gather3(n_nodes, n_idx):
  per_w = n_idx // (NC * NS)
  nchunk = per_w // CH
  assert nchunk * CH == per_w
  mesh = plsc.VectorSubcoreMesh(core_axis_name="c", subcore_axis_name="s",
                                num_cores=NC, num_subcores=NS)

  @functools.partial(
      pl.kernel,
      out_type=tuple(jax.ShapeDtypeStruct((n_idx, DG), jnp.float32)
                     for _ in range(3)),
      mesh=mesh,
      scratch_types=[
          pltpu.VMEM((CH,), jnp.int32),
          pltpu.VMEM((CH, DG), jnp.float32),
          pltpu.SemaphoreType.DMA,
      ],
  )
  def gat(idx, t0, t1, t2, o0, o1, o2, idxb, buf, sem):
    c = lax.axis_index("c")
    s = lax.axis_index("s")
    wid = s * NC + c

    def chunk(i, carry):
      base = wid * per_w + i * CH
      pltpu.sync_copy(idx.at[pl.ds(base, CH)], idxb)
      for tab, o in ((t0, o0), (t1, o1), (t2, o2)):
        pltpu.async_copy(tab.at[idxb], buf, sem).wait()
        pltpu.sync_copy(buf, o.at[pl.ds(base, CH)])
      return carry
    lax.fori_loop(0, nchunk, chunk, 0)

  return gat


def _tc_scale(upd, vals):
  m = upd.shape[0]
  blk = 8192
  grid = m // blk
  assert grid * blk == m

  def body(u_r, v_r, o_r):
    o_r[...] = u_r[...] * v_r[...]

  return pl.pallas_call(
      body,
      grid=(grid,),
      in_specs=[pl.BlockSpec((blk, DG), lambda i: (i, 0)),
                pl.BlockSpec((blk, 1), lambda i: (i, 0))],
      out_specs=pl.BlockSpec((blk, DG), lambda i: (i, 0)),
      out_shape=jax.ShapeDtypeStruct((m, DG), jnp.float32),
  )(upd, vals.reshape(-1, 1))


def _dg(x, w):
  return lax.dot_general(x, w, (((1,), (1,)), ((), ())),
                         preferred_element_type=jnp.float32)


def _tc_fused(lf, f, wa, wb, ba):
  n = f.shape[0]
  blk = 2000
  grid = n // blk

  def body(lf_r, f_r, wa_r, wb_r, ba_r, p1_r, iw_r):
    l, x = lf_r[...], f_r[...]
    p1_r[...] = _dg(l + x, wa_r[...]) + ba_r[...]
    iw_r[...] = _dg(l * x, wb_r[...])

  full = lambda a: pl.BlockSpec(a.shape, lambda i: (0, 0))
  row = pl.BlockSpec((blk, DG), lambda i: (i, 0))
  return pl.pallas_call(
      body,
      grid=(grid,),
      in_specs=[row, row, full(wa), full(wb), full(ba)],
      out_specs=[row, row],
      out_shape=[jax.ShapeDtypeStruct((n, DG), jnp.float32)] * 2,
  )(lf, f, wa, wb, ba)


def _tc_relu_add(p1, s2, b):
  n = p1.shape[0]
  blk = 2000
  grid = n // blk

  def body(p_r, s_r, b_r, o_r):
    o_r[...] = jnp.maximum(p_r[...] + s_r[...] + b_r[...], 0.0)

  full = lambda a: pl.BlockSpec(a.shape, lambda i: (0, 0))
  row = pl.BlockSpec((blk, DG), lambda i: (i, 0))
  return pl.pallas_call(
      body,
      grid=(grid,),
      in_specs=[row, row, full(b)],
      out_specs=row,
      out_shape=jax.ShapeDtypeStruct((n, DG), jnp.float32),
  )(p1, s2, b)


def _tc_mlp(gs, wu, wi, b1, t2w, b2, t3p):
  batch = gs[0].shape[0] // 2
  blk = 512
  nblk = batch // blk

  def body(g0u, g1u, g2u, g0i, g1i, g2i,
           wu0, wu1, wu2, wi0, wi1, wi2,
           b1_r, t2_r, b2_r, t3_r, out_r):
    h = b1_r[...] + _dg(g0u[...], wu0[...]) + _dg(g1u[...], wu1[...]) \
        + _dg(g2u[...], wu2[...]) \
        + _dg(g0i[...], wi0[...]) + _dg(g1i[...], wi1[...]) \
        + _dg(g2i[...], wi2[...])
    h = jnp.maximum(h, 0.0)
    h2 = _dg(h, t2_r[...]) + b2_r[...]
    out_r[...] = jnp.dot(h2, t3_r[...],
                         preferred_element_type=jnp.float32)

  full = lambda a: pl.BlockSpec(a.shape, lambda i: (0, 0))
  rowu = lambda a: pl.BlockSpec((blk, a.shape[1]), lambda i: (i, 0))
  rowi = lambda a: pl.BlockSpec((blk, a.shape[1]), lambda i: (i + nblk, 0))
  return pl.pallas_call(
      body,
      grid=(nblk,),
      in_specs=[rowu(g) for g in gs] + [rowi(g) for g in gs]
               + [full(w) for w in wu] + [full(w) for w in wi]
               + [full(b1), full(t2w), full(b2), full(t3p)],
      out_specs=pl.BlockSpec((blk, DG), lambda i: (i, 0)),
      out_shape=jax.ShapeDtypeStruct((batch, DG), jnp.float32),
  )(*gs, *gs, *wu, *wi, b1, t2w, b2, t3p)


def _pad2(a, r, c):
  return jnp.pad(a, ((0, r - a.shape[0]), (0, c - a.shape[1])))


def kernel(userIdx, itemIdx, rows, cols, L_vals, uEmbd, iEmbd,
           W1a, b1a, W1b, b1b, W2a, b2a, W2b, b2b,
           T1w, T1b, T2w, T2b, T3w, T3b):
  n_u, d0 = uEmbd.shape
  n_i = iEmbd.shape[0]
  n = n_u + n_i
  assert n_u == n_i
  e = rows.shape[0]
  eh = e // 2

  ept = -(-(eh // NS) // CH) * CH
  e_side = ept * NS
  pad = e_side - eh
  r32 = rows.astype(jnp.int32)
  c32 = cols.astype(jnp.int32)
  v32 = L_vals.astype(jnp.float32)
  zi = jnp.zeros((pad,), jnp.int32)
  zf = jnp.zeros((pad,), jnp.float32)
  colsp = jnp.concatenate([c32[:eh], zi, c32[eh:], zi])
  rowsp = jnp.concatenate([r32[:eh], zi, r32[eh:], zi])
  valsp = jnp.concatenate([v32[:eh], zf, v32[eh:], zf])

  d1 = W1a.shape[0]
  d2 = W2a.shape[0]
  d0p = -(-d0 // LN) * LN
  d1p = -(-d1 // LN) * LN
  dA = 64

  f0p = _pad2(jnp.concatenate([uEmbd, iEmbd], axis=0), n, DG)

  gs112 = _make_gather_scale(n, d0p, 2 * e_side)
  gs80 = _make_gather_scale(n, d1p, 2 * e_side)
  gs64 = _make_gather_scale(n, dA, 2 * e_side)

  def spmm(gsc, feat):
    upd = gsc(feat, colsp, valsp)
    return jnp.zeros((n, DG), jnp.float32).at[rowsp].add(upd)

  bpad = lambda b: jnp.pad(b, (0, DG - b.shape[0])).reshape(1, -1)

  lf = spmm(gs112, f0p)
  p1, iw = _tc_fused(lf, f0p, _pad2(W1a, DG, DG), _pad2(W1b, DG, DG),
                     bpad(b1a))
  s2 = spmm(gs80, iw)
  f2 = _tc_relu_add(p1, s2, bpad(b1b))

  lf2 = spmm(gs80, f2)
  p12, iw2 = _tc_fused(lf2, f2, _pad2(W2a, DG, DG), _pad2(W2b, DG, DG),
                       bpad(b2a))
  s22 = spmm(gs64, iw2)
  f3 = _tc_relu_add(p12, s22, bpad(b2b))

  idx = jnp.concatenate([userIdx, itemIdx]).astype(jnp.int32)
  g0, g2, g3 = _make_gather3(n, idx.shape[0])(idx, f0p, f2, f3)

  demb = d0 + d1 + d2

  def t1split(off):
    a = _pad2(T1w[:, off:off + d0], T1b.shape[0], DG)
    b = _pad2(T1w[:, off + d0:off + d0 + d1], T1b.shape[0], DG)
    c2 = _pad2(T1w[:, off + d0 + d1:off + demb], T1b.shape[0], DG)
    return a, b, c2

  t3p = _pad2(T3w.reshape(-1, 1), T2w.shape[0], DG)
  y = _tc_mlp((g0, g2, g3), t1split(0), t1split(demb),
              T1b.reshape(1, -1), T2w, T2b.reshape(1, -1), t3p)
  return y[:, 0] + T3b[0]
